# Initial kernel scaffold; baseline (speedup 1.0000x reference)
import jax
import jax.numpy as jnp
from jax.experimental import pallas as pl


def kernel(x, table, W, b):
    raise NotImplementedError("write your pallas kernel here")


# speedup vs baseline: 3.1896x; 3.1896x over previous
import functools

import jax
import jax.numpy as jnp
from jax import lax
from jax.experimental import pallas as pl
from jax.experimental.pallas import tpu as pltpu
from jax.experimental.pallas import tpu_sc as plsc

_B, _L, _D = 16384, 200, 64
_NB = 1000

_NC, _NS = 2, 16
_NW = _NC * _NS
_BPW = _B // _NW
_CHUNK = 128
_NCHUNK = _BPW // _CHUNK
_H0, _H1 = 104, 96
_LANES = 16
_DV = _D // _LANES
_RUNROLL = 8
_NSLOT = 6


def _pool_body(x_hbm, table_hbm, out_hbm, idx_v, rows_v, pool_v, *sems):
    wid = lax.axis_index("s") * _NC + lax.axis_index("c")
    base = wid * _BPW
    inv_l = 1.0 / _L

    def issue(i, slot):
        pltpu.async_copy(table_hbm.at[idx_v.at[i, pl.ds(0, _H0)]],
                         rows_v.at[slot, pl.ds(0, _H0)], sems[slot])
        pltpu.async_copy(table_hbm.at[idx_v.at[i, pl.ds(_H0, _H1)]],
                         rows_v.at[slot, pl.ds(_H0, _H1)], sems[slot])

    def drain(slot):
        pltpu.make_async_copy(table_hbm.at[pl.ds(0, _H0)],
                              rows_v.at[slot, pl.ds(0, _H0)],
                              sems[slot]).wait()
        pltpu.make_async_copy(table_hbm.at[pl.ds(0, _H1)],
                              rows_v.at[slot, pl.ds(_H0, _H1)],
                              sems[slot]).wait()

    def accum_store(i, slot):
        acc = tuple(jnp.zeros((_LANES,), jnp.float32) for _ in range(_DV))

        def row(j, acc):
            jb = j * _RUNROLL
            out = []
            for k in range(_DV):
                r = [rows_v[slot, jb + u, pl.ds(k * _LANES, _LANES)]
                     for u in range(_RUNROLL)]
                s01 = (r[0] + r[1]) + (r[2] + r[3])
                s23 = (r[4] + r[5]) + (r[6] + r[7])
                out.append(acc[k] + (s01 + s23))
            return tuple(out)

        acc = lax.fori_loop(0, _L // _RUNROLL, row, acc)
        for k in range(_DV):
            pool_v[i, pl.ds(k * _LANES, _LANES)] = acc[k] * inv_l

    def chunk_body(c, _):
        cb = base + c * _CHUNK
        pltpu.sync_copy(x_hbm.at[pl.ds(cb, _CHUNK)], idx_v)
        for s in range(_NSLOT - 1):
            issue(s, s)

        def group_body(t, _):
            b0 = _NSLOT * t
            for s in range(_NSLOT):
                nxt = b0 + s + _NSLOT - 1

                @pl.when(nxt < _CHUNK)
                def _():
                    issue(nxt, (s + _NSLOT - 1) % _NSLOT)

                drain(s)
                accum_store(b0 + s, s)
            return 0

        lax.fori_loop(0, _CHUNK // _NSLOT, group_body, 0)
        for s in range(_CHUNK % _NSLOT):
            i = (_CHUNK // _NSLOT) * _NSLOT + s
            drain(i % _NSLOT)
            accum_store(i, i % _NSLOT)
        pltpu.sync_copy(pool_v, out_hbm.at[pl.ds(cb, _CHUNK)])
        return 0

    lax.fori_loop(0, _NCHUNK, chunk_body, 0)


_pool = functools.partial(
    pl.kernel,
    mesh=plsc.VectorSubcoreMesh(core_axis_name="c", subcore_axis_name="s"),
    out_type=jax.ShapeDtypeStruct((_B, _D), jnp.float32),
    scratch_types=[
        pltpu.VMEM((_CHUNK, _L), jnp.int32),
        pltpu.VMEM((_NSLOT, _L, _D), jnp.float32),
        pltpu.VMEM((_CHUNK, _D), jnp.float32),
    ] + [pltpu.SemaphoreType.DMA] * _NSLOT,
    compiler_params=pltpu.CompilerParams(use_tc_tiling_on_sc=False),
)(_pool_body)


def _mm_body(p_ref, w_ref, b_ref, o_ref):
    o_ref[...] = lax.dot_general(
        p_ref[...], w_ref[...], (((1,), (1,)), ((), ())),
        preferred_element_type=jnp.float32) + b_ref[...]


_BM = 1024
_mm = pl.pallas_call(
    _mm_body,
    grid=(_B // _BM,),
    in_specs=[
        pl.BlockSpec((_BM, _D), lambda i: (i, 0)),
        pl.BlockSpec((_NB, _D), lambda i: (0, 0)),
        pl.BlockSpec((1, _NB), lambda i: (0, 0)),
    ],
    out_specs=pl.BlockSpec((_BM, _NB), lambda i: (i, 0)),
    out_shape=jax.ShapeDtypeStruct((_B, _NB), jnp.float32),
)


def kernel(x, table, W, b):
    pooled = _pool(x.astype(jnp.int32), table)
    return _mm(pooled, W, b.reshape(1, _NB))
